# baseline (speedup 1.0000x reference)
import jax
import jax.numpy as jnp
from jax.experimental import pallas as pl


def _body(feat_ref, pe_ref, out_ref):
    out_ref[...] = feat_ref[...] + pe_ref[...][None]


def kernel(feat_tokens, pos_emb):
    B, P, D = feat_tokens.shape
    BB = 128
    return pl.pallas_call(
        _body,
        grid=(B // BB,),
        in_specs=[
            pl.BlockSpec((BB, P, D), lambda i: (i, 0, 0)),
            pl.BlockSpec((P, D), lambda i: (0, 0)),
        ],
        out_specs=pl.BlockSpec((BB, P, D), lambda i: (i, 0, 0)),
        out_shape=jax.ShapeDtypeStruct((B, P, D), feat_tokens.dtype),
    )(feat_tokens, pos_emb)
